# baseline (speedup 1.0000x reference)
import functools

import jax
import jax.numpy as jnp
from jax import lax
from jax.experimental import pallas as pl
from jax.experimental.pallas import tpu as pltpu
from jax.experimental.pallas import tpu_sc as plsc

_NUM_SELECT = 300
_BN = 2000

_PARTS = 4
_BINS = 512
_CAPSC = 128
_NEG_INF = float("-inf")
_INT_MIN = -(2**31)


def _fused_body(logits_ref, posmap_ref, m_ref):
    pm = posmap_ref[...]
    s = jnp.sum(pm, axis=1, keepdims=True)
    pm = pm / jnp.where(s == 0.0, 1.0, s)

    x = logits_ref[0]
    p = jax.nn.sigmoid(x)
    prob = jax.lax.dot_general(
        p, pm, (((1,), (1,)), ((), ())), preferred_element_type=jnp.float32
    )
    m_ref[0, 0] = jnp.max(prob.reshape(8, _BN // 8, prob.shape[-1]), axis=2)


def _fused_rowmax(pred_logits, positive_map):
    B, N, T = pred_logits.shape
    C = positive_map.shape[0]
    grid = (B, N // _BN)
    return pl.pallas_call(
        _fused_body,
        grid=grid,
        in_specs=[
            pl.BlockSpec((1, _BN, T), lambda b, i: (b, i, 0)),
            pl.BlockSpec((C, T), lambda b, i: (0, 0)),
        ],
        out_specs=pl.BlockSpec((1, 1, 8, _BN // 8), lambda b, i: (b, i, 0, 0)),
        out_shape=jax.ShapeDtypeStruct((B, N // _BN, 8, _BN // 8), jnp.float32),
    )(pred_logits, positive_map)


def _cand_body(logits_ref, posmap_ref, mv_ref, prob_ref):
    pm = posmap_ref[...]
    s = jnp.sum(pm, axis=1, keepdims=True)
    pm = pm / jnp.where(s == 0.0, 1.0, s)
    x = logits_ref[0]
    p = jax.nn.sigmoid(x)
    prob = jax.lax.dot_general(
        p, pm, (((1,), (1,)), ((), ())), preferred_element_type=jnp.float32
    )
    valid = mv_ref[0] > _NEG_INF
    prob_ref[0] = jnp.where(valid, prob, _NEG_INF)


def _cand_probs(sel_logits, positive_map, mv):
    B, RP, T = sel_logits.shape
    C = positive_map.shape[0]
    return pl.pallas_call(
        _cand_body,
        grid=(B,),
        in_specs=[
            pl.BlockSpec((1, RP, T), lambda b: (b, 0, 0)),
            pl.BlockSpec((C, T), lambda b: (0, 0)),
            pl.BlockSpec((1, RP, 1), lambda b: (b, 0, 0)),
        ],
        out_specs=pl.BlockSpec((1, RP, C), lambda b: (b, 0, 0)),
        out_shape=jax.ShapeDtypeStruct((B, RP, C), jnp.float32),
    )(sel_logits, positive_map, mv.reshape(B, RP, 1))


def _find_threshold_bin(comb, lane, target):
    def fbody(jj, carry):
        acc, beta, found = carry
        j = 31 - jj
        v = comb[pl.ds(j * 16, 16)]
        srev = jnp.flip(v, 0)
        cs = plsc.cumsum(srev)
        tot = jnp.max(cs)
        hit = (acc + cs) >= target
        anym = jnp.max(hit.astype(jnp.int32)) > 0
        k = jnp.min(jnp.where(hit, lane, 16))
        bcand = j * 16 + 15 - k
        take = jnp.logical_and(jnp.logical_not(found), anym)
        beta = jnp.where(take, bcand, beta)
        found = jnp.logical_or(found, anym)
        acc = acc + jnp.where(found, 0, tot)
        return acc, beta, found

    acc, beta, found = lax.fori_loop(
        0, 32, fbody, (jnp.int32(0), jnp.int32(0), jnp.bool_(False))
    )
    return found, beta


def _count_above(comb, lane, beta):
    def abody(j, acc):
        v = comb[pl.ds(j * 16, 16)]
        bins = j * 16 + lane
        return acc + jnp.sum(jnp.where(bins > beta, v, 0))

    return lax.fori_loop(0, 32, abody, jnp.int32(0))


def _make_select(B, M, target):
    assert M % (_PARTS * 16) == 0
    chunk = M // _PARTS
    nvec_full = M // 16
    nvec = chunk // 16
    nb_per_core = B // 2
    mesh = plsc.VectorSubcoreMesh(core_axis_name="c", subcore_axis_name="s")

    @functools.partial(
        pl.kernel,
        out_type=[
            jax.ShapeDtypeStruct((B * _PARTS * _CAPSC,), jnp.float32),
            jax.ShapeDtypeStruct((B * _PARTS * _CAPSC,), jnp.int32),
        ],
        mesh=mesh,
        compiler_params=pltpu.CompilerParams(needs_layout_passes=False),
        scratch_types=[
            pltpu.VMEM((chunk,), jnp.float32),
            pltpu.VMEM((16 * _BINS,), jnp.int32),
            pltpu.VMEM((_BINS,), jnp.int32),
            pltpu.VMEM((_BINS,), jnp.int32),
            pltpu.VMEM((_PARTS, _BINS), jnp.int32),
            pltpu.VMEM((_CAPSC + 16,), jnp.float32),
            pltpu.VMEM((_CAPSC + 16,), jnp.int32),
            pltpu.VMEM_SHARED((16, _BINS), jnp.int32),
        ],
    )
    def select(vals_hbm, vals_out, idx_out, vbuf, histf, comb, comb1,
               tmp4, valbuf, idxbuf, sh_hists):
        c = lax.axis_index("c")
        s = lax.axis_index("s")
        bic = s // _PARTS
        p = s % _PARTS
        b = c * nb_per_core + bic
        lane = lax.iota(jnp.int32, 16)
        ones = jnp.ones((16,), jnp.int32)
        zeros = jnp.zeros((16,), jnp.int32)

        pltpu.sync_copy(vals_hbm.at[pl.ds(b * M + p * chunk, chunk)], vbuf)

        def zero_hist(i, _):
            for u in range(8):
                histf[pl.ds(i * 128 + u * 16, 16)] = zeros
            return 0

        def make_combine(dst):
            def combine_local(j, _):
                acc = zeros
                for l in range(16):
                    acc = acc + histf[pl.ds(l * _BINS + j * 16, 16)]
                dst[pl.ds(j * 16, 16)] = acc
                return 0

            return combine_local

        lax.fori_loop(0, _BINS // 8, zero_hist, 0)

        def h1body(i, _):
            for u in range(4):
                v = vbuf[pl.ds(i * 64 + u * 16, 16)]
                bits = plsc.bitcast(v, jnp.int32)
                bidx = jnp.clip((bits >> 18) - 3552, 0, _BINS - 1)
                plsc.addupdate_scatter(histf, [lane * _BINS + bidx], ones)
            return 0

        lax.fori_loop(0, nvec // 4, h1body, 0)
        lax.fori_loop(0, _BINS // 16, make_combine(comb), 0)
        pltpu.sync_copy(comb, sh_hists.at[s])
        plsc.subcore_barrier()
        pltpu.sync_copy(sh_hists.at[pl.ds(bic * _PARTS, _PARTS)], tmp4)

        def cb1(j, _):
            acc = (
                tmp4[0, pl.ds(j * 16, 16)]
                + tmp4[1, pl.ds(j * 16, 16)]
                + tmp4[2, pl.ds(j * 16, 16)]
                + tmp4[3, pl.ds(j * 16, 16)]
            )
            comb1[pl.ds(j * 16, 16)] = acc
            return 0

        lax.fori_loop(0, _BINS // 16, cb1, 0)
        plsc.subcore_barrier()
        found, beta = _find_threshold_bin(comb1, lane, target)
        ok = jnp.logical_and(found, beta > 0)
        t1 = jnp.where(ok, (beta + 3552) << 18, jnp.int32(_INT_MIN))

        lax.fori_loop(0, _BINS // 8, zero_hist, 0)

        def h2body(i, _):
            for u in range(4):
                v = vbuf[pl.ds(i * 64 + u * 16, 16)]
                bits = plsc.bitcast(v, jnp.int32)
                inbin = jnp.logical_and(bits >= t1, bits < t1 + (1 << 18))
                bidx = (bits >> 9) & (_BINS - 1)
                plsc.addupdate_scatter(
                    histf, [lane * _BINS + bidx], ones, mask=inbin
                )
            return 0

        lax.fori_loop(0, nvec // 4, h2body, 0)
        lax.fori_loop(0, _BINS // 16, make_combine(comb), 0)
        pltpu.sync_copy(comb, sh_hists.at[s])
        plsc.subcore_barrier()
        pltpu.sync_copy(sh_hists.at[pl.ds(bic * _PARTS, _PARTS)], tmp4)

        def cb2(j, _):
            acc = (
                tmp4[0, pl.ds(j * 16, 16)]
                + tmp4[1, pl.ds(j * 16, 16)]
                + tmp4[2, pl.ds(j * 16, 16)]
                + tmp4[3, pl.ds(j * 16, 16)]
            )
            comb[pl.ds(j * 16, 16)] = acc
            return 0

        lax.fori_loop(0, _BINS // 16, cb2, 0)
        above = _count_above(comb1, lane, beta)
        found2, beta2 = _find_threshold_bin(comb, lane, target - above)
        t2 = jnp.where(
            ok,
            t1 + jnp.where(found2, beta2 << 9, 0),
            jnp.int32(_INT_MIN),
        )

        neg = jnp.full((16,), _NEG_INF, jnp.float32)

        def prefill(i, _):
            valbuf[pl.ds(i * 16, 16)] = neg
            idxbuf[pl.ds(i * 16, 16)] = zeros
            return 0

        lax.fori_loop(0, (_CAPSC + 16) // 16, prefill, 0)

        def pbody(i, off):
            v = vbuf[pl.ds(i * 16, 16)]
            bits = plsc.bitcast(v, jnp.int32)
            msk = bits >= t2
            gidx = p * chunk + i * 16 + lane
            plsc.store_compressed(valbuf.at[pl.ds(off, 16)], v, mask=msk)
            plsc.store_compressed(idxbuf.at[pl.ds(off, 16)], gidx, mask=msk)
            cnt = jnp.max(plsc.all_reduce_population_count(msk))
            return jnp.minimum(off + cnt, _CAPSC)

        lax.fori_loop(0, nvec, pbody, jnp.int32(0))

        r = (b * _PARTS + p) * _CAPSC
        pltpu.sync_copy(valbuf.at[pl.ds(0, _CAPSC)], vals_out.at[pl.ds(r, _CAPSC)])
        pltpu.sync_copy(idxbuf.at[pl.ds(0, _CAPSC)], idx_out.at[pl.ds(r, _CAPSC)])

    return select


def kernel(pred_logits, pred_boxes, positive_map, target_sizes):
    B, N, T = pred_logits.shape
    C = positive_map.shape[0]

    m = _fused_rowmax(pred_logits, positive_map)
    m = m.reshape(B, N)

    MP = 20480
    mpad = jnp.pad(m, ((0, 0), (0, MP - N)), constant_values=_NEG_INF)
    sel_q = _make_select(B, MP, _NUM_SELECT)
    mv, mi = sel_q(mpad.reshape(-1))
    W = _PARTS * _CAPSC
    mv = mv.reshape(B, W)
    mi = mi.reshape(B, W)

    glog = jnp.take_along_axis(pred_logits, mi[:, :, None], axis=1)
    cand = _cand_probs(glog, positive_map, mv)
    MC = W * C
    sel_c = _make_select(B, MC, _NUM_SELECT)
    cv, ci_ = sel_c(cand.reshape(-1))
    cv = cv.reshape(B, W)
    ci_ = ci_.reshape(B, W)
    scores, cpos = lax.top_k(cv, _NUM_SELECT)
    ci = jnp.take_along_axis(ci_, cpos, axis=1)
    labels = ci % C
    topk_boxes = jnp.take_along_axis(mi, ci // C, axis=1)

    gb = jnp.take_along_axis(pred_boxes, topk_boxes[:, :, None], axis=1)
    cx, cy, w, h = gb[..., 0], gb[..., 1], gb[..., 2], gb[..., 3]
    xyxy = jnp.stack(
        [cx - 0.5 * w, cy - 0.5 * h, cx + 0.5 * w, cy + 0.5 * h], axis=-1
    )
    img_h = target_sizes[:, 0].astype(jnp.float32)
    img_w = target_sizes[:, 1].astype(jnp.float32)
    scale_fct = jnp.stack([img_w, img_h, img_w, img_h], axis=1)
    boxes = xyxy * scale_fct[:, None, :]
    return scores, labels, boxes


# speedup vs baseline: 1.0304x; 1.0304x over previous
import functools

import jax
import jax.numpy as jnp
from jax import lax
from jax.experimental import pallas as pl
from jax.experimental.pallas import tpu as pltpu
from jax.experimental.pallas import tpu_sc as plsc

_NUM_SELECT = 300
_BN = 2000

_PARTS = 4
_BINS = 512
_CAPSC = 128
_NEG_INF = float("-inf")
_INT_MIN = -(2**31)


def _fused_body(logits_ref, posmap_ref, m_ref):
    pm = posmap_ref[...]
    s = jnp.sum(pm, axis=1, keepdims=True)
    pm = pm / jnp.where(s == 0.0, 1.0, s)

    x = logits_ref[0]
    p = jax.nn.sigmoid(x)
    prob = jax.lax.dot_general(
        p, pm, (((1,), (1,)), ((), ())), preferred_element_type=jnp.float32
    )
    m_ref[0, 0] = jnp.max(prob.reshape(8, _BN // 8, prob.shape[-1]), axis=2)


def _fused_rowmax(pred_logits, positive_map):
    B, N, T = pred_logits.shape
    C = positive_map.shape[0]
    grid = (B, N // _BN)
    return pl.pallas_call(
        _fused_body,
        grid=grid,
        in_specs=[
            pl.BlockSpec((1, _BN, T), lambda b, i: (b, i, 0)),
            pl.BlockSpec((C, T), lambda b, i: (0, 0)),
        ],
        out_specs=pl.BlockSpec((1, 1, 8, _BN // 8), lambda b, i: (b, i, 0, 0)),
        out_shape=jax.ShapeDtypeStruct((B, N // _BN, 8, _BN // 8), jnp.float32),
    )(pred_logits, positive_map)


def _cand_body(logits_ref, posmap_ref, prob_ref):
    pm = posmap_ref[...]
    s = jnp.sum(pm, axis=1, keepdims=True)
    pm = pm / jnp.where(s == 0.0, 1.0, s)
    x = logits_ref[0]
    p = jax.nn.sigmoid(x)
    prob = jax.lax.dot_general(
        p, pm, (((1,), (1,)), ((), ())), preferred_element_type=jnp.float32
    )
    row = jax.lax.broadcasted_iota(jnp.int32, prob.shape, 0)
    prob_ref[0] = jnp.where(row < _NUM_SELECT, prob, _NEG_INF)


def _cand_probs(sel_logits, positive_map):
    B, RP, T = sel_logits.shape
    C = positive_map.shape[0]
    return pl.pallas_call(
        _cand_body,
        grid=(B,),
        in_specs=[
            pl.BlockSpec((1, RP, T), lambda b: (b, 0, 0)),
            pl.BlockSpec((C, T), lambda b: (0, 0)),
        ],
        out_specs=pl.BlockSpec((1, RP, C), lambda b: (b, 0, 0)),
        out_shape=jax.ShapeDtypeStruct((B, RP, C), jnp.float32),
    )(sel_logits, positive_map)


def _find_threshold_bin(comb, lane, target):
    def fbody(jj, carry):
        acc, beta, found = carry
        j = 31 - jj
        v = comb[pl.ds(j * 16, 16)]
        srev = jnp.flip(v, 0)
        cs = plsc.cumsum(srev)
        tot = jnp.max(cs)
        hit = (acc + cs) >= target
        anym = jnp.max(hit.astype(jnp.int32)) > 0
        k = jnp.min(jnp.where(hit, lane, 16))
        bcand = j * 16 + 15 - k
        take = jnp.logical_and(jnp.logical_not(found), anym)
        beta = jnp.where(take, bcand, beta)
        found = jnp.logical_or(found, anym)
        acc = acc + jnp.where(found, 0, tot)
        return acc, beta, found

    acc, beta, found = lax.fori_loop(
        0, 32, fbody, (jnp.int32(0), jnp.int32(0), jnp.bool_(False))
    )
    return found, beta


def _count_above(comb, lane, beta):
    def abody(j, acc):
        v = comb[pl.ds(j * 16, 16)]
        bins = j * 16 + lane
        return acc + jnp.sum(jnp.where(bins > beta, v, 0))

    return lax.fori_loop(0, 32, abody, jnp.int32(0))


def _make_select(B, M, target):
    assert M % (_PARTS * 16) == 0
    chunk = M // _PARTS
    nvec_full = M // 16
    nvec = chunk // 16
    nb_per_core = B // 2
    mesh = plsc.VectorSubcoreMesh(core_axis_name="c", subcore_axis_name="s")

    @functools.partial(
        pl.kernel,
        out_type=[
            jax.ShapeDtypeStruct((B * _PARTS * _CAPSC,), jnp.float32),
            jax.ShapeDtypeStruct((B * _PARTS * _CAPSC,), jnp.int32),
        ],
        mesh=mesh,
        compiler_params=pltpu.CompilerParams(needs_layout_passes=False),
        scratch_types=[
            pltpu.VMEM((chunk,), jnp.float32),
            pltpu.VMEM((16 * _BINS,), jnp.int32),
            pltpu.VMEM((_BINS,), jnp.int32),
            pltpu.VMEM((_BINS,), jnp.int32),
            pltpu.VMEM((_PARTS, _BINS), jnp.int32),
            pltpu.VMEM((_CAPSC + 16,), jnp.float32),
            pltpu.VMEM((_CAPSC + 16,), jnp.int32),
            pltpu.VMEM_SHARED((16, _BINS), jnp.int32),
        ],
    )
    def select(vals_hbm, vals_out, idx_out, vbuf, histf, comb, comb1,
               tmp4, valbuf, idxbuf, sh_hists):
        c = lax.axis_index("c")
        s = lax.axis_index("s")
        bic = s // _PARTS
        p = s % _PARTS
        b = c * nb_per_core + bic
        lane = lax.iota(jnp.int32, 16)
        ones = jnp.ones((16,), jnp.int32)
        zeros = jnp.zeros((16,), jnp.int32)

        pltpu.sync_copy(vals_hbm.at[pl.ds(b * M + p * chunk, chunk)], vbuf)

        def zero_hist(i, _):
            for u in range(8):
                histf[pl.ds(i * 128 + u * 16, 16)] = zeros
            return 0

        def make_combine(dst):
            def combine_local(j, _):
                acc = zeros
                for l in range(16):
                    acc = acc + histf[pl.ds(l * _BINS + j * 16, 16)]
                dst[pl.ds(j * 16, 16)] = acc
                return 0

            return combine_local

        lax.fori_loop(0, _BINS // 8, zero_hist, 0)

        def h1body(i, _):
            for u in range(4):
                v = vbuf[pl.ds(i * 64 + u * 16, 16)]
                bits = plsc.bitcast(v, jnp.int32)
                bidx = jnp.clip((bits >> 18) - 3552, 0, _BINS - 1)
                plsc.addupdate_scatter(histf, [lane * _BINS + bidx], ones)
            return 0

        lax.fori_loop(0, nvec // 4, h1body, 0)
        lax.fori_loop(0, _BINS // 16, make_combine(comb), 0)
        pltpu.sync_copy(comb, sh_hists.at[s])
        plsc.subcore_barrier()
        pltpu.sync_copy(sh_hists.at[pl.ds(bic * _PARTS, _PARTS)], tmp4)

        def cb1(j, _):
            acc = (
                tmp4[0, pl.ds(j * 16, 16)]
                + tmp4[1, pl.ds(j * 16, 16)]
                + tmp4[2, pl.ds(j * 16, 16)]
                + tmp4[3, pl.ds(j * 16, 16)]
            )
            comb1[pl.ds(j * 16, 16)] = acc
            return 0

        lax.fori_loop(0, _BINS // 16, cb1, 0)
        plsc.subcore_barrier()
        found, beta = _find_threshold_bin(comb1, lane, target)
        ok = jnp.logical_and(found, beta > 0)
        t1 = jnp.where(ok, (beta + 3552) << 18, jnp.int32(_INT_MIN))

        lax.fori_loop(0, _BINS // 8, zero_hist, 0)

        def h2body(i, _):
            for u in range(4):
                v = vbuf[pl.ds(i * 64 + u * 16, 16)]
                bits = plsc.bitcast(v, jnp.int32)
                inbin = jnp.logical_and(bits >= t1, bits < t1 + (1 << 18))
                bidx = (bits >> 9) & (_BINS - 1)
                plsc.addupdate_scatter(
                    histf, [lane * _BINS + bidx], ones, mask=inbin
                )
            return 0

        lax.fori_loop(0, nvec // 4, h2body, 0)
        lax.fori_loop(0, _BINS // 16, make_combine(comb), 0)
        pltpu.sync_copy(comb, sh_hists.at[s])
        plsc.subcore_barrier()
        pltpu.sync_copy(sh_hists.at[pl.ds(bic * _PARTS, _PARTS)], tmp4)

        def cb2(j, _):
            acc = (
                tmp4[0, pl.ds(j * 16, 16)]
                + tmp4[1, pl.ds(j * 16, 16)]
                + tmp4[2, pl.ds(j * 16, 16)]
                + tmp4[3, pl.ds(j * 16, 16)]
            )
            comb[pl.ds(j * 16, 16)] = acc
            return 0

        lax.fori_loop(0, _BINS // 16, cb2, 0)
        above = _count_above(comb1, lane, beta)
        found2, beta2 = _find_threshold_bin(comb, lane, target - above)
        t2 = jnp.where(
            ok,
            t1 + jnp.where(found2, beta2 << 9, 0),
            jnp.int32(_INT_MIN),
        )

        neg = jnp.full((16,), _NEG_INF, jnp.float32)

        def prefill(i, _):
            valbuf[pl.ds(i * 16, 16)] = neg
            idxbuf[pl.ds(i * 16, 16)] = zeros
            return 0

        lax.fori_loop(0, (_CAPSC + 16) // 16, prefill, 0)

        def pbody(i, off):
            v = vbuf[pl.ds(i * 16, 16)]
            bits = plsc.bitcast(v, jnp.int32)
            msk = bits >= t2
            gidx = p * chunk + i * 16 + lane
            plsc.store_compressed(valbuf.at[pl.ds(off, 16)], v, mask=msk)
            plsc.store_compressed(idxbuf.at[pl.ds(off, 16)], gidx, mask=msk)
            cnt = jnp.max(plsc.all_reduce_population_count(msk))
            return jnp.minimum(off + cnt, _CAPSC)

        lax.fori_loop(0, nvec, pbody, jnp.int32(0))

        r = (b * _PARTS + p) * _CAPSC
        pltpu.sync_copy(valbuf.at[pl.ds(0, _CAPSC)], vals_out.at[pl.ds(r, _CAPSC)])
        pltpu.sync_copy(idxbuf.at[pl.ds(0, _CAPSC)], idx_out.at[pl.ds(r, _CAPSC)])

    return select


def kernel(pred_logits, pred_boxes, positive_map, target_sizes):
    B, N, T = pred_logits.shape
    C = positive_map.shape[0]

    m = _fused_rowmax(pred_logits, positive_map)
    m = m.reshape(B, N)

    MP = 20480
    mpad = jnp.pad(m, ((0, 0), (0, MP - N)), constant_values=_NEG_INF)
    sel_q = _make_select(B, MP, _NUM_SELECT)
    mv, mi = sel_q(mpad.reshape(-1))
    W = _PARTS * _CAPSC
    mv = mv.reshape(B, W)
    mi = mi.reshape(B, W)
    _, qpos = lax.top_k(mv, _NUM_SELECT)
    qsel = jnp.take_along_axis(mi, qpos, axis=1)
    qsel = jnp.sort(qsel, axis=1)

    RP = 304
    qsel_pad = jnp.pad(qsel, ((0, 0), (0, RP - _NUM_SELECT)))
    glog = jnp.take_along_axis(
        pred_logits, qsel_pad[:, :, None], axis=1
    )
    cand = _cand_probs(glog, positive_map)
    MC = 24576
    cflat = jnp.pad(
        cand.reshape(B, RP * C),
        ((0, 0), (0, MC - RP * C)),
        constant_values=_NEG_INF,
    )
    sel_c = _make_select(B, MC, _NUM_SELECT)
    cv, ci_ = sel_c(cflat.reshape(-1))
    cv = cv.reshape(B, W)
    ci_ = ci_.reshape(B, W)
    scores, cpos = lax.top_k(cv, _NUM_SELECT)
    ci = jnp.take_along_axis(ci_, cpos, axis=1)
    labels = ci % C
    topk_boxes = jnp.take_along_axis(qsel, ci // C, axis=1)

    gb = jnp.take_along_axis(pred_boxes, topk_boxes[:, :, None], axis=1)
    cx, cy, w, h = gb[..., 0], gb[..., 1], gb[..., 2], gb[..., 3]
    xyxy = jnp.stack(
        [cx - 0.5 * w, cy - 0.5 * h, cx + 0.5 * w, cy + 0.5 * h], axis=-1
    )
    img_h = target_sizes[:, 0].astype(jnp.float32)
    img_w = target_sizes[:, 1].astype(jnp.float32)
    scale_fct = jnp.stack([img_w, img_h, img_w, img_h], axis=1)
    boxes = xyxy * scale_fct[:, None, :]
    return scores, labels, boxes


# speedup vs baseline: 1.1707x; 1.1362x over previous
import functools

import jax
import jax.numpy as jnp
from jax import lax
from jax.experimental import pallas as pl
from jax.experimental.pallas import tpu as pltpu
from jax.experimental.pallas import tpu_sc as plsc

_NUM_SELECT = 300
_BN = 4000

_PARTS = 4
_BINS = 512
_CAPSC = 128
_NEG_INF = float("-inf")
_INT_MIN = -(2**31)


def _fused_body(logits_ref, posmap_ref, m_ref):
    pm = posmap_ref[...]
    s = jnp.sum(pm, axis=1, keepdims=True)
    pm = pm / jnp.where(s == 0.0, 1.0, s)

    x = logits_ref[0]
    p = jax.nn.sigmoid(x)
    prob = jax.lax.dot_general(
        p, pm, (((1,), (1,)), ((), ())), preferred_element_type=jnp.float32
    )
    m_ref[0, 0] = jnp.max(prob.reshape(8, _BN // 8, prob.shape[-1]), axis=2)


def _fused_rowmax(pred_logits, positive_map):
    B, N, T = pred_logits.shape
    C = positive_map.shape[0]
    grid = (B, N // _BN)
    return pl.pallas_call(
        _fused_body,
        grid=grid,
        in_specs=[
            pl.BlockSpec((1, _BN, T), lambda b, i: (b, i, 0)),
            pl.BlockSpec((C, T), lambda b, i: (0, 0)),
        ],
        out_specs=pl.BlockSpec((1, 1, 8, _BN // 8), lambda b, i: (b, i, 0, 0)),
        out_shape=jax.ShapeDtypeStruct((B, N // _BN, 8, _BN // 8), jnp.float32),
    )(pred_logits, positive_map)


def _cand_body(logits_ref, posmap_ref, prob_ref):
    pm = posmap_ref[...]
    s = jnp.sum(pm, axis=1, keepdims=True)
    pm = pm / jnp.where(s == 0.0, 1.0, s)
    x = logits_ref[0]
    p = jax.nn.sigmoid(x)
    prob = jax.lax.dot_general(
        p, pm, (((1,), (1,)), ((), ())), preferred_element_type=jnp.float32
    )
    row = jax.lax.broadcasted_iota(jnp.int32, prob.shape, 0)
    prob_ref[0] = jnp.where(row < _NUM_SELECT, prob, _NEG_INF)


def _cand_probs(sel_logits, positive_map):
    B, RP, T = sel_logits.shape
    C = positive_map.shape[0]
    return pl.pallas_call(
        _cand_body,
        grid=(B,),
        in_specs=[
            pl.BlockSpec((1, RP, T), lambda b: (b, 0, 0)),
            pl.BlockSpec((C, T), lambda b: (0, 0)),
        ],
        out_specs=pl.BlockSpec((1, RP, C), lambda b: (b, 0, 0)),
        out_shape=jax.ShapeDtypeStruct((B, RP, C), jnp.float32),
    )(sel_logits, positive_map)


def _find_threshold_bin(comb, lane, target):
    def fbody(jj, carry):
        acc, beta, found = carry
        j = 31 - jj
        v = comb[pl.ds(j * 16, 16)]
        srev = jnp.flip(v, 0)
        cs = plsc.cumsum(srev)
        tot = jnp.max(cs)
        hit = (acc + cs) >= target
        anym = jnp.max(hit.astype(jnp.int32)) > 0
        k = jnp.min(jnp.where(hit, lane, 16))
        bcand = j * 16 + 15 - k
        take = jnp.logical_and(jnp.logical_not(found), anym)
        beta = jnp.where(take, bcand, beta)
        found = jnp.logical_or(found, anym)
        acc = acc + jnp.where(found, 0, tot)
        return acc, beta, found

    acc, beta, found = lax.fori_loop(
        0, 32, fbody, (jnp.int32(0), jnp.int32(0), jnp.bool_(False))
    )
    return found, beta


def _count_above(comb, lane, beta):
    def abody(j, acc):
        v = comb[pl.ds(j * 16, 16)]
        bins = j * 16 + lane
        return acc + jnp.sum(jnp.where(bins > beta, v, 0))

    return lax.fori_loop(0, 32, abody, jnp.int32(0))


def _make_select(B, M, target):
    assert M % (_PARTS * 16) == 0
    chunk = M // _PARTS
    nvec_full = M // 16
    nvec = chunk // 16
    nb_per_core = B // 2
    mesh = plsc.VectorSubcoreMesh(core_axis_name="c", subcore_axis_name="s")

    @functools.partial(
        pl.kernel,
        out_type=[
            jax.ShapeDtypeStruct((B * _PARTS * _CAPSC,), jnp.float32),
            jax.ShapeDtypeStruct((B * _PARTS * _CAPSC,), jnp.int32),
        ],
        mesh=mesh,
        compiler_params=pltpu.CompilerParams(needs_layout_passes=False),
        scratch_types=[
            pltpu.VMEM((chunk,), jnp.float32),
            pltpu.VMEM((16 * _BINS,), jnp.int32),
            pltpu.VMEM((_BINS,), jnp.int32),
            pltpu.VMEM((_BINS,), jnp.int32),
            pltpu.VMEM((_PARTS, _BINS), jnp.int32),
            pltpu.VMEM((_CAPSC + 16,), jnp.float32),
            pltpu.VMEM((_CAPSC + 16,), jnp.int32),
            pltpu.VMEM_SHARED((16, _BINS), jnp.int32),
        ],
    )
    def select(vals_hbm, vals_out, idx_out, vbuf, histf, comb, comb1,
               tmp4, valbuf, idxbuf, sh_hists):
        c = lax.axis_index("c")
        s = lax.axis_index("s")
        bic = s // _PARTS
        p = s % _PARTS
        b = c * nb_per_core + bic
        lane = lax.iota(jnp.int32, 16)
        ones = jnp.ones((16,), jnp.int32)
        zeros = jnp.zeros((16,), jnp.int32)

        pltpu.sync_copy(vals_hbm.at[pl.ds(b * M + p * chunk, chunk)], vbuf)

        def zero_hist(i, _):
            for u in range(8):
                histf[pl.ds(i * 128 + u * 16, 16)] = zeros
            return 0

        def make_combine(dst):
            def combine_local(j, _):
                acc = zeros
                for l in range(16):
                    acc = acc + histf[pl.ds(l * _BINS + j * 16, 16)]
                dst[pl.ds(j * 16, 16)] = acc
                return 0

            return combine_local

        lax.fori_loop(0, _BINS // 8, zero_hist, 0)

        def h1body(i, _):
            for u in range(4):
                v = vbuf[pl.ds(i * 64 + u * 16, 16)]
                bits = plsc.bitcast(v, jnp.int32)
                bidx = jnp.clip((bits >> 18) - 3552, 0, _BINS - 1)
                plsc.addupdate_scatter(histf, [lane * _BINS + bidx], ones)
            return 0

        lax.fori_loop(0, nvec // 4, h1body, 0)
        lax.fori_loop(0, _BINS // 16, make_combine(comb), 0)
        pltpu.sync_copy(comb, sh_hists.at[s])
        plsc.subcore_barrier()
        pltpu.sync_copy(sh_hists.at[pl.ds(bic * _PARTS, _PARTS)], tmp4)

        def cb1(j, _):
            acc = (
                tmp4[0, pl.ds(j * 16, 16)]
                + tmp4[1, pl.ds(j * 16, 16)]
                + tmp4[2, pl.ds(j * 16, 16)]
                + tmp4[3, pl.ds(j * 16, 16)]
            )
            comb1[pl.ds(j * 16, 16)] = acc
            return 0

        lax.fori_loop(0, _BINS // 16, cb1, 0)
        plsc.subcore_barrier()
        found, beta = _find_threshold_bin(comb1, lane, target)
        ok = jnp.logical_and(found, beta > 0)
        t1 = jnp.where(ok, (beta + 3552) << 18, jnp.int32(_INT_MIN))

        lax.fori_loop(0, _BINS // 8, zero_hist, 0)

        def h2body(i, _):
            for u in range(4):
                v = vbuf[pl.ds(i * 64 + u * 16, 16)]
                bits = plsc.bitcast(v, jnp.int32)
                inbin = jnp.logical_and(bits >= t1, bits < t1 + (1 << 18))
                bidx = (bits >> 9) & (_BINS - 1)
                plsc.addupdate_scatter(
                    histf, [lane * _BINS + bidx], ones, mask=inbin
                )
            return 0

        lax.fori_loop(0, nvec // 4, h2body, 0)
        lax.fori_loop(0, _BINS // 16, make_combine(comb), 0)
        pltpu.sync_copy(comb, sh_hists.at[s])
        plsc.subcore_barrier()
        pltpu.sync_copy(sh_hists.at[pl.ds(bic * _PARTS, _PARTS)], tmp4)

        def cb2(j, _):
            acc = (
                tmp4[0, pl.ds(j * 16, 16)]
                + tmp4[1, pl.ds(j * 16, 16)]
                + tmp4[2, pl.ds(j * 16, 16)]
                + tmp4[3, pl.ds(j * 16, 16)]
            )
            comb[pl.ds(j * 16, 16)] = acc
            return 0

        lax.fori_loop(0, _BINS // 16, cb2, 0)
        above = _count_above(comb1, lane, beta)
        found2, beta2 = _find_threshold_bin(comb, lane, target - above)
        t2 = jnp.where(
            ok,
            t1 + jnp.where(found2, beta2 << 9, 0),
            jnp.int32(_INT_MIN),
        )

        neg = jnp.full((16,), _NEG_INF, jnp.float32)

        def prefill(i, _):
            valbuf[pl.ds(i * 16, 16)] = neg
            idxbuf[pl.ds(i * 16, 16)] = zeros
            return 0

        lax.fori_loop(0, (_CAPSC + 16) // 16, prefill, 0)

        def pbody(i, off):
            v = vbuf[pl.ds(i * 16, 16)]
            bits = plsc.bitcast(v, jnp.int32)
            msk = bits >= t2
            gidx = p * chunk + i * 16 + lane
            plsc.store_compressed(valbuf.at[pl.ds(off, 16)], v, mask=msk)
            plsc.store_compressed(idxbuf.at[pl.ds(off, 16)], gidx, mask=msk)
            cnt = jnp.max(plsc.all_reduce_population_count(msk))
            return jnp.minimum(off + cnt, _CAPSC)

        lax.fori_loop(0, nvec, pbody, jnp.int32(0))

        r = (b * _PARTS + p) * _CAPSC
        pltpu.sync_copy(valbuf.at[pl.ds(0, _CAPSC)], vals_out.at[pl.ds(r, _CAPSC)])
        pltpu.sync_copy(idxbuf.at[pl.ds(0, _CAPSC)], idx_out.at[pl.ds(r, _CAPSC)])

    return select


def kernel(pred_logits, pred_boxes, positive_map, target_sizes):
    B, N, T = pred_logits.shape
    C = positive_map.shape[0]

    m = _fused_rowmax(pred_logits, positive_map)
    m = m.reshape(B, N)

    MP = 20480
    mpad = jnp.pad(m, ((0, 0), (0, MP - N)), constant_values=_NEG_INF)
    sel_q = _make_select(B, MP, _NUM_SELECT)
    mv, mi = sel_q(mpad.reshape(-1))
    W = _PARTS * _CAPSC
    mv = mv.reshape(B, W)
    mi = mi.reshape(B, W)
    _, qpos = lax.top_k(mv, _NUM_SELECT)
    qsel = jnp.take_along_axis(mi, qpos, axis=1)
    qsel = jnp.sort(qsel, axis=1)

    RP = 304
    qsel_pad = jnp.pad(qsel, ((0, 0), (0, RP - _NUM_SELECT)))
    glog = jnp.take_along_axis(
        pred_logits, qsel_pad[:, :, None], axis=1
    )
    cand = _cand_probs(glog, positive_map)
    MC = 24576
    cflat = jnp.pad(
        cand.reshape(B, RP * C),
        ((0, 0), (0, MC - RP * C)),
        constant_values=_NEG_INF,
    )
    sel_c = _make_select(B, MC, _NUM_SELECT)
    cv, ci_ = sel_c(cflat.reshape(-1))
    cv = cv.reshape(B, W)
    ci_ = ci_.reshape(B, W)
    scores, cpos = lax.top_k(cv, _NUM_SELECT)
    ci = jnp.take_along_axis(ci_, cpos, axis=1)
    labels = ci % C
    topk_boxes = jnp.take_along_axis(qsel, ci // C, axis=1)

    gb = jnp.take_along_axis(pred_boxes, topk_boxes[:, :, None], axis=1)
    cx, cy, w, h = gb[..., 0], gb[..., 1], gb[..., 2], gb[..., 3]
    xyxy = jnp.stack(
        [cx - 0.5 * w, cy - 0.5 * h, cx + 0.5 * w, cy + 0.5 * h], axis=-1
    )
    img_h = target_sizes[:, 0].astype(jnp.float32)
    img_w = target_sizes[:, 1].astype(jnp.float32)
    scale_fct = jnp.stack([img_w, img_h, img_w, img_h], axis=1)
    boxes = xyxy * scale_fct[:, None, :]
    return scores, labels, boxes
